# baseline (speedup 1.0000x reference)
import functools

import jax
import jax.numpy as jnp
from jax import lax
from jax.experimental import pallas as pl
from jax.experimental.pallas import tpu as pltpu
from jax.experimental.pallas import tpu_sc as plsc

NC = 2
NS = 16
NW = NC * NS
CHUNK = 128
LANES = 16


def _sc_mesh():
    return plsc.VectorSubcoreMesh(
        core_axis_name="c", subcore_axis_name="s", num_cores=NC,
        num_subcores=NS)


def _make_deg_kernel(npad, n_chunks):
    rows_per_tile = npad // NS

    @functools.partial(
        pl.kernel,
        out_type=jax.ShapeDtypeStruct((NC, npad), jnp.float32),
        mesh=_sc_mesh(),
        scratch_types=[
            pltpu.VMEM((n_chunks, CHUNK), jnp.int32),
            pltpu.VMEM((CHUNK,), jnp.float32),
            pltpu.VMEM((rows_per_tile,), jnp.float32),
            pltpu.VMEM_SHARED((npad,), jnp.float32),
        ],
    )
    def deg_kernel(dst_hbm, out_hbm, idx_v, ones_v, init_v, acc_sh):
        c = lax.axis_index("c")
        s = lax.axis_index("s")
        w = s * NC + c
        base = s * rows_per_tile

        ones_vec = jnp.full((LANES,), 1.0, dtype=jnp.float32)
        init_val = jnp.where(c == 0, 1.0, 0.0)
        init_vec = jnp.full((LANES,), 1.0, dtype=jnp.float32) * init_val

        def fill_ones(i, _):
            ones_v[pl.ds(i * LANES, LANES)] = ones_vec
            return 0
        lax.fori_loop(0, CHUNK // LANES, fill_ones, 0)

        def fill_init(i, _):
            init_v[pl.ds(i * LANES, LANES)] = init_vec
            return 0
        lax.fori_loop(0, rows_per_tile // LANES, fill_init, 0)

        pltpu.sync_copy(init_v, acc_sh.at[pl.ds(base, rows_per_tile)])
        pltpu.sync_copy(dst_hbm.at[w], idx_v)
        plsc.subcore_barrier()

        def body(j, _):
            pltpu.sync_copy(ones_v, acc_sh.at[idx_v.at[j]], add=True)
            return 0
        lax.fori_loop(0, n_chunks, body, 0)

        plsc.subcore_barrier()
        pltpu.sync_copy(acc_sh.at[pl.ds(base, rows_per_tile)],
                        out_hbm.at[c, pl.ds(base, rows_per_tile)])

    return deg_kernel


def _make_edge_scatter_kernel(n, npad, dg, d, n_chunks, chunk=CHUNK,
                              depth=2, dtype=jnp.float32, sc_tiling=False,
                              stage_src=False):
    rows_per_tile = npad // NS
    assert n_chunks > depth
    assert n % NS == 0 or not stage_src
    cpr = CHUNK // chunk
    n_irows = n_chunks // cpr

    @functools.partial(
        pl.kernel,
        out_type=jax.ShapeDtypeStruct((NC, npad, d), dtype),
        mesh=_sc_mesh(),
        compiler_params=pltpu.CompilerParams(
            use_tc_tiling_on_sc=not sc_tiling),
        scratch_types=[
            pltpu.VMEM((n_irows, CHUNK), jnp.int32),
            pltpu.VMEM((n_irows, CHUNK), jnp.int32),
            pltpu.VMEM((depth, chunk, dg), dtype),
            pltpu.VMEM_SHARED((npad, d), dtype),
        ] + ([pltpu.VMEM_SHARED((n, dg), dtype)] if stage_src else []) + [
            pltpu.SemaphoreType.DMA,
            pltpu.SemaphoreType.DMA,
        ],
    )
    def scat_kernel(v_hbm, src_hbm, dst_hbm, zrows_hbm, out_hbm,
                    sidx_v, didx_v, rows_v, acc_sh, *rest):
        if stage_src:
            v_sh, sem_g, sem_s = rest
        else:
            sem_g, sem_s = rest
            v_sh = None
        c = lax.axis_index("c")
        s = lax.axis_index("s")
        w = s * NC + c
        base = s * rows_per_tile

        def idx(buf, j):
            if cpr == 1:
                return buf.at[j]
            return buf.at[j // cpr, pl.ds((j % cpr) * chunk, chunk)]

        pltpu.sync_copy(zrows_hbm, acc_sh.at[pl.ds(base, rows_per_tile)])
        pltpu.sync_copy(src_hbm.at[w], sidx_v)
        pltpu.sync_copy(dst_hbm.at[w], didx_v)
        if stage_src:
            vrows = n // NS
            pltpu.sync_copy(v_hbm.at[pl.ds(s * vrows, vrows)],
                            v_sh.at[pl.ds(s * vrows, vrows)])
        plsc.subcore_barrier()
        vsrc = v_sh if stage_src else v_hbm

        def gather(j, slot):
            pltpu.async_copy(vsrc.at[idx(sidx_v, j)], rows_v.at[slot],
                             sem_g)

        def wait_gather(j, slot):
            pltpu.make_async_copy(vsrc.at[idx(sidx_v, j)],
                                  rows_v.at[slot], sem_g).wait()

        def src_rows(slot):
            if d == dg:
                return rows_v.at[slot]
            return rows_v.at[slot, :, pl.ds(0, d)]

        def scatter(j, slot):
            pltpu.async_copy(src_rows(slot), acc_sh.at[idx(didx_v, j)],
                             sem_s, add=True)

        def wait_scatter(j, slot):
            pltpu.make_async_copy(src_rows(slot),
                                  acc_sh.at[idx(didx_v, j)], sem_s).wait()

        for k in range(depth - 1):
            gather(k, k)
        wait_gather(0, 0)
        scatter(0, 0)
        gather(depth - 1, depth - 1)

        def body(j, _):
            slot = lax.rem(j, depth)
            wait_gather(j, slot)
            scatter(j, slot)
            wait_scatter(j - 1, lax.rem(j - 1, depth))
            gather(j + depth - 1, lax.rem(j + depth - 1, depth))
            return 0
        lax.fori_loop(1, n_chunks - depth + 1, body, 0)

        for j in range(n_chunks - depth + 1, n_chunks):
            wait_gather(j, j % depth)
            scatter(j, j % depth)
            wait_scatter(j - 1, (j - 1) % depth)
        wait_scatter(n_chunks - 1, (n_chunks - 1) % depth)

        plsc.subcore_barrier()
        pltpu.sync_copy(acc_sh.at[pl.ds(base, rows_per_tile)],
                        out_hbm.at[c, pl.ds(base, rows_per_tile)])

    return scat_kernel


def _mm_kernel(x_ref, w_ref, u_ref):
    u_ref[...] = jnp.dot(x_ref[...], w_ref[...],
                         preferred_element_type=jnp.float32)


def _scale_kernel(d0_ref, d1_ref, u_ref, v_ref, dinv_ref):
    dinv = lax.rsqrt(d0_ref[0] + d1_ref[0])
    v_ref[...] = u_ref[...] * dinv
    dinv_ref[...] = dinv


def _mid_kernel(p0_ref, p1_ref, v_ref, dinv_ref, b_ref, w_ref, o_ref):
    dinv = dinv_ref[...]
    agg = (p0_ref[0].astype(jnp.float32) + p1_ref[0].astype(jnp.float32)
           + v_ref[...].astype(jnp.float32))
    h = jax.nn.relu(agg * dinv + b_ref[...])
    v2 = jnp.dot(h, w_ref[...], preferred_element_type=jnp.float32) * dinv
    pad = o_ref.shape[1] - v2.shape[1]
    if pad:
        v2 = jnp.concatenate(
            [v2, jnp.zeros((v2.shape[0], pad), jnp.float32)], axis=1)
    o_ref[...] = v2.astype(o_ref.dtype)


def _final_kernel(q0_ref, q1_ref, v_ref, dinv_ref, b_ref, o_ref):
    d_out = o_ref.shape[1]
    agg = (q0_ref[0, :, :d_out].astype(jnp.float32)
           + q1_ref[0, :, :d_out].astype(jnp.float32)
           + v_ref[:, :d_out].astype(jnp.float32))
    o_ref[...] = agg * dinv_ref[...] + b_ref[...]


def kernel(x, edge_index, W1, b1, W2, b2):
    n, d_in = x.shape
    d_h = W1.shape[1]
    d_out = W2.shape[1]
    e = edge_index.shape[1]

    npad = ((n + 16 * NS - 1) // (16 * NS)) * (16 * NS)
    n_pad_rows = npad - n if npad > n else NS
    n_chunks = (e + NW * CHUNK - 1) // (NW * CHUNK)
    epad = NW * n_chunks * CHUNK

    src = edge_index[0]
    dst = edge_index[1]
    pad = epad - e
    pad_i = jnp.arange(pad, dtype=jnp.int32)
    src_p = jnp.concatenate([src, pad_i % n]).reshape(NW, n_chunks, CHUNK)
    dst_p = jnp.concatenate(
        [dst, n + pad_i % jnp.int32(max(npad - n, 1))]
    ).reshape(NW, n_chunks, CHUNK)

    deg_k = _make_deg_kernel(npad, n_chunks)
    degp = deg_k(dst_p)

    bn = 400 if n % 400 == 0 else 8
    grid = (n // bn,)
    degc = degp.reshape(NC, npad, 1)
    col_spec = pl.BlockSpec((bn, 1), lambda i: (i, 0))
    deg_spec0 = pl.BlockSpec((1, bn, 1), lambda i: (0, i, 0))
    deg_spec1 = pl.BlockSpec((1, bn, 1), lambda i: (1, i, 0))
    u1 = pl.pallas_call(
        _mm_kernel,
        grid=grid,
        in_specs=[
            pl.BlockSpec((bn, d_in), lambda i: (i, 0)),
            pl.BlockSpec((d_in, d_h), lambda i: (0, 0)),
        ],
        out_specs=pl.BlockSpec((bn, d_h), lambda i: (i, 0)),
        out_shape=jax.ShapeDtypeStruct((n, d_h), jnp.float32),
    )(x, W1)
    v1, dinv = pl.pallas_call(
        _scale_kernel,
        grid=grid,
        in_specs=[
            deg_spec0, deg_spec1,
            pl.BlockSpec((bn, d_h), lambda i: (i, 0)),
        ],
        out_specs=[
            pl.BlockSpec((bn, d_h), lambda i: (i, 0)),
            col_spec,
        ],
        out_shape=[
            jax.ShapeDtypeStruct((n, d_h), jnp.float32),
            jax.ShapeDtypeStruct((n, 1), jnp.float32),
        ],
    )(degc, degc, u1)

    scat1 = _make_edge_scatter_kernel(n, npad, d_h, d_h, 4 * n_chunks,
                                      chunk=32, depth=8)
    zrows1 = jnp.zeros((npad // NS, d_h), jnp.float32)
    part1 = scat1(v1, src_p, dst_p, zrows1)

    d2p = d_out
    part_spec0 = pl.BlockSpec((1, bn, d_h), lambda i: (0, i, 0))
    part_spec1 = pl.BlockSpec((1, bn, d_h), lambda i: (1, i, 0))
    v2p = pl.pallas_call(
        _mid_kernel,
        grid=grid,
        in_specs=[
            part_spec0, part_spec1,
            pl.BlockSpec((bn, d_h), lambda i: (i, 0)),
            col_spec,
            pl.BlockSpec((1, d_h), lambda i: (0, 0)),
            pl.BlockSpec((d_h, d_out), lambda i: (0, 0)),
        ],
        out_specs=pl.BlockSpec((bn, d2p), lambda i: (i, 0)),
        out_shape=jax.ShapeDtypeStruct((n, d2p), jnp.float32),
    )(part1, part1, v1, dinv, b1.reshape(1, d_h), W2)

    scat2 = _make_edge_scatter_kernel(n, npad, d2p, d2p, 2 * n_chunks,
                                      chunk=64, depth=8, sc_tiling=True)
    zrows2 = jnp.zeros((npad // NS, d2p), jnp.float32)
    part2 = scat2(v2p, src_p, dst_p, zrows2)

    part2_spec0 = pl.BlockSpec((1, bn, d2p), lambda i: (0, i, 0))
    part2_spec1 = pl.BlockSpec((1, bn, d2p), lambda i: (1, i, 0))
    z = pl.pallas_call(
        _final_kernel,
        grid=grid,
        in_specs=[
            part2_spec0, part2_spec1,
            pl.BlockSpec((bn, d2p), lambda i: (i, 0)),
            col_spec,
            pl.BlockSpec((1, d_out), lambda i: (0, 0)),
        ],
        out_specs=pl.BlockSpec((bn, d_out), lambda i: (i, 0)),
        out_shape=jax.ShapeDtypeStruct((n, d_out), jnp.float32),
    )(part2, part2, v2p, dinv, b2.reshape(1, d_out))

    return z


# speedup vs baseline: 1.0371x; 1.0371x over previous
import functools

import jax
import jax.numpy as jnp
from jax import lax
from jax.experimental import pallas as pl
from jax.experimental.pallas import tpu as pltpu
from jax.experimental.pallas import tpu_sc as plsc

NC = 2
NS = 16
NW = NC * NS
CHUNK = 128
LANES = 16


def _sc_mesh():
    return plsc.VectorSubcoreMesh(
        core_axis_name="c", subcore_axis_name="s", num_cores=NC,
        num_subcores=NS)


def _make_deg_kernel(npad, n_chunks):
    rows_per_tile = npad // NS

    @functools.partial(
        pl.kernel,
        out_type=jax.ShapeDtypeStruct((NC, npad), jnp.float32),
        mesh=_sc_mesh(),
        scratch_types=[
            pltpu.VMEM((n_chunks, CHUNK), jnp.int32),
            pltpu.VMEM((CHUNK,), jnp.float32),
            pltpu.VMEM((rows_per_tile,), jnp.float32),
            pltpu.VMEM_SHARED((npad,), jnp.float32),
        ],
    )
    def deg_kernel(dst_hbm, out_hbm, idx_v, ones_v, init_v, acc_sh):
        c = lax.axis_index("c")
        s = lax.axis_index("s")
        w = s * NC + c
        base = s * rows_per_tile

        ones_vec = jnp.full((LANES,), 1.0, dtype=jnp.float32)
        init_val = jnp.where(c == 0, 1.0, 0.0)
        init_vec = jnp.full((LANES,), 1.0, dtype=jnp.float32) * init_val

        def fill_ones(i, _):
            ones_v[pl.ds(i * LANES, LANES)] = ones_vec
            return 0
        lax.fori_loop(0, CHUNK // LANES, fill_ones, 0)

        def fill_init(i, _):
            init_v[pl.ds(i * LANES, LANES)] = init_vec
            return 0
        lax.fori_loop(0, rows_per_tile // LANES, fill_init, 0)

        pltpu.sync_copy(init_v, acc_sh.at[pl.ds(base, rows_per_tile)])
        pltpu.sync_copy(dst_hbm.at[w], idx_v)
        plsc.subcore_barrier()

        def body(j, _):
            pltpu.sync_copy(ones_v, acc_sh.at[idx_v.at[j]], add=True)
            return 0
        lax.fori_loop(0, n_chunks, body, 0)

        plsc.subcore_barrier()
        pltpu.sync_copy(acc_sh.at[pl.ds(base, rows_per_tile)],
                        out_hbm.at[c, pl.ds(base, rows_per_tile)])

    return deg_kernel


def _make_edge_scatter_kernel(n, npad, dg, d, n_chunks, chunk=CHUNK,
                              depth=2, dtype=jnp.float32, sc_tiling=False,
                              stage_src=False):
    rows_per_tile = npad // NS
    assert n_chunks > depth
    assert n % NS == 0 or not stage_src
    cpr = CHUNK // chunk
    n_irows = n_chunks // cpr

    @functools.partial(
        pl.kernel,
        out_type=jax.ShapeDtypeStruct((NC, npad, d), dtype),
        mesh=_sc_mesh(),
        compiler_params=pltpu.CompilerParams(
            use_tc_tiling_on_sc=not sc_tiling),
        scratch_types=[
            pltpu.VMEM((n_irows, CHUNK), jnp.int32),
            pltpu.VMEM((n_irows, CHUNK), jnp.int32),
            pltpu.VMEM((depth, chunk, dg), dtype),
            pltpu.VMEM_SHARED((npad, d), dtype),
        ] + ([pltpu.VMEM_SHARED((n, dg), dtype)] if stage_src else []) + [
            pltpu.SemaphoreType.DMA,
            pltpu.SemaphoreType.DMA,
        ],
    )
    def scat_kernel(v_hbm, src_hbm, dst_hbm, zrows_hbm, out_hbm,
                    sidx_v, didx_v, rows_v, acc_sh, *rest):
        if stage_src:
            v_sh, sem_g, sem_s = rest
        else:
            sem_g, sem_s = rest
            v_sh = None
        c = lax.axis_index("c")
        s = lax.axis_index("s")
        w = s * NC + c
        base = s * rows_per_tile

        def idx(buf, j):
            if cpr == 1:
                return buf.at[j]
            return buf.at[j // cpr, pl.ds((j % cpr) * chunk, chunk)]

        pltpu.sync_copy(zrows_hbm, acc_sh.at[pl.ds(base, rows_per_tile)])
        pltpu.sync_copy(src_hbm.at[w], sidx_v)
        pltpu.sync_copy(dst_hbm.at[w], didx_v)
        if stage_src:
            vrows = n // NS
            pltpu.sync_copy(v_hbm.at[pl.ds(s * vrows, vrows)],
                            v_sh.at[pl.ds(s * vrows, vrows)])
        plsc.subcore_barrier()
        vsrc = v_sh if stage_src else v_hbm

        def gather(j, slot):
            pltpu.async_copy(vsrc.at[idx(sidx_v, j)], rows_v.at[slot],
                             sem_g)

        def wait_gather(j, slot):
            pltpu.make_async_copy(vsrc.at[idx(sidx_v, j)],
                                  rows_v.at[slot], sem_g).wait()

        def src_rows(slot):
            if d == dg:
                return rows_v.at[slot]
            return rows_v.at[slot, :, pl.ds(0, d)]

        def scatter(j, slot):
            pltpu.async_copy(src_rows(slot), acc_sh.at[idx(didx_v, j)],
                             sem_s, add=True)

        def wait_scatter(j, slot):
            pltpu.make_async_copy(src_rows(slot),
                                  acc_sh.at[idx(didx_v, j)], sem_s).wait()

        for k in range(depth - 1):
            gather(k, k)
        wait_gather(0, 0)
        scatter(0, 0)
        gather(depth - 1, depth - 1)

        def body(j, _):
            slot = lax.rem(j, depth)
            wait_gather(j, slot)
            scatter(j, slot)
            wait_scatter(j - 1, lax.rem(j - 1, depth))
            gather(j + depth - 1, lax.rem(j + depth - 1, depth))
            return 0
        lax.fori_loop(1, n_chunks - depth + 1, body, 0)

        for j in range(n_chunks - depth + 1, n_chunks):
            wait_gather(j, j % depth)
            scatter(j, j % depth)
            wait_scatter(j - 1, (j - 1) % depth)
        wait_scatter(n_chunks - 1, (n_chunks - 1) % depth)

        plsc.subcore_barrier()
        pltpu.sync_copy(acc_sh.at[pl.ds(base, rows_per_tile)],
                        out_hbm.at[c, pl.ds(base, rows_per_tile)])

    return scat_kernel


def _mm_scale_kernel(d0_ref, d1_ref, x_ref, w_ref, v_ref, dinv_ref):
    dinv = lax.rsqrt(d0_ref[0] + d1_ref[0])
    xw = jnp.dot(x_ref[...], w_ref[...], preferred_element_type=jnp.float32)
    v_ref[...] = (xw * dinv).astype(v_ref.dtype)
    dinv_ref[...] = dinv


def _mid_kernel(p0_ref, p1_ref, v_ref, dinv_ref, b_ref, w_ref, o_ref):
    dinv = dinv_ref[...]
    agg = (p0_ref[0].astype(jnp.float32) + p1_ref[0].astype(jnp.float32)
           + v_ref[...].astype(jnp.float32))
    h = jax.nn.relu(agg * dinv + b_ref[...])
    v2 = jnp.dot(h, w_ref[...], preferred_element_type=jnp.float32) * dinv
    pad = o_ref.shape[1] - v2.shape[1]
    if pad:
        v2 = jnp.concatenate(
            [v2, jnp.zeros((v2.shape[0], pad), jnp.float32)], axis=1)
    o_ref[...] = v2.astype(o_ref.dtype)


def _final_kernel(q0_ref, q1_ref, v_ref, dinv_ref, b_ref, o_ref):
    d_out = o_ref.shape[1]
    agg = (q0_ref[0, :, :d_out].astype(jnp.float32)
           + q1_ref[0, :, :d_out].astype(jnp.float32)
           + v_ref[:, :d_out].astype(jnp.float32))
    o_ref[...] = agg * dinv_ref[...] + b_ref[...]


def kernel(x, edge_index, W1, b1, W2, b2):
    n, d_in = x.shape
    d_h = W1.shape[1]
    d_out = W2.shape[1]
    e = edge_index.shape[1]

    npad = ((n + 16 * NS - 1) // (16 * NS)) * (16 * NS)
    n_pad_rows = npad - n if npad > n else NS
    n_chunks = (e + NW * CHUNK - 1) // (NW * CHUNK)
    epad = NW * n_chunks * CHUNK

    src = edge_index[0]
    dst = edge_index[1]
    pad = epad - e
    pad_i = jnp.arange(pad, dtype=jnp.int32)
    src_p = jnp.concatenate([src, pad_i % n]).reshape(NW, n_chunks, CHUNK)
    dst_p = jnp.concatenate(
        [dst, n + pad_i % jnp.int32(max(npad - n, 1))]
    ).reshape(NW, n_chunks, CHUNK)

    deg_k = _make_deg_kernel(npad, n_chunks)
    degp = deg_k(dst_p)

    bn = 400 if n % 400 == 0 else 8
    grid = (n // bn,)
    degc = degp.reshape(NC, npad, 1)
    col_spec = pl.BlockSpec((bn, 1), lambda i: (i, 0))
    deg_spec0 = pl.BlockSpec((1, bn, 1), lambda i: (0, i, 0))
    deg_spec1 = pl.BlockSpec((1, bn, 1), lambda i: (1, i, 0))
    v1, dinv = pl.pallas_call(
        _mm_scale_kernel,
        grid=grid,
        in_specs=[
            deg_spec0, deg_spec1,
            pl.BlockSpec((bn, d_in), lambda i: (i, 0)),
            pl.BlockSpec((d_in, d_h), lambda i: (0, 0)),
        ],
        out_specs=[
            pl.BlockSpec((bn, d_h), lambda i: (i, 0)),
            col_spec,
        ],
        out_shape=[
            jax.ShapeDtypeStruct((n, d_h), jnp.float32),
            jax.ShapeDtypeStruct((n, 1), jnp.float32),
        ],
    )(degc, degc, x, W1)

    scat1 = _make_edge_scatter_kernel(n, npad, d_h, d_h, 4 * n_chunks,
                                      chunk=32, depth=8)
    zrows1 = jnp.zeros((npad // NS, d_h), jnp.float32)
    part1 = scat1(v1, src_p, dst_p, zrows1)

    d2p = d_out
    part_spec0 = pl.BlockSpec((1, bn, d_h), lambda i: (0, i, 0))
    part_spec1 = pl.BlockSpec((1, bn, d_h), lambda i: (1, i, 0))
    v2p = pl.pallas_call(
        _mid_kernel,
        grid=grid,
        in_specs=[
            part_spec0, part_spec1,
            pl.BlockSpec((bn, d_h), lambda i: (i, 0)),
            col_spec,
            pl.BlockSpec((1, d_h), lambda i: (0, 0)),
            pl.BlockSpec((d_h, d_out), lambda i: (0, 0)),
        ],
        out_specs=pl.BlockSpec((bn, d2p), lambda i: (i, 0)),
        out_shape=jax.ShapeDtypeStruct((n, d2p), jnp.float32),
    )(part1, part1, v1, dinv, b1.reshape(1, d_h), W2)

    scat2 = _make_edge_scatter_kernel(n, npad, d2p, d2p, 2 * n_chunks,
                                      chunk=64, depth=8, sc_tiling=True)
    zrows2 = jnp.zeros((npad // NS, d2p), jnp.float32)
    part2 = scat2(v2p, src_p, dst_p, zrows2)

    part2_spec0 = pl.BlockSpec((1, bn, d2p), lambda i: (0, i, 0))
    part2_spec1 = pl.BlockSpec((1, bn, d2p), lambda i: (1, i, 0))
    z = pl.pallas_call(
        _final_kernel,
        grid=grid,
        in_specs=[
            part2_spec0, part2_spec1,
            pl.BlockSpec((bn, d2p), lambda i: (i, 0)),
            col_spec,
            pl.BlockSpec((1, d_out), lambda i: (0, 0)),
        ],
        out_specs=pl.BlockSpec((bn, d_out), lambda i: (i, 0)),
        out_shape=jax.ShapeDtypeStruct((n, d_out), jnp.float32),
    )(part2, part2, v2p, dinv, b2.reshape(1, d_out))

    return z
